# baseline (speedup 1.0000x reference)
import functools

import jax
import jax.numpy as jnp
from jax.experimental import pallas as pl
from jax.experimental.pallas import tpu as pltpu

_SMOOTHING = 0.1
_CONFIDENCE = 1.0 - _SMOOTHING


def _rowsum(x, ones):
    return jax.lax.dot_general(x, ones, (((1,), (0,)), ((), ())),
                               preferred_element_type=jnp.float32)


def _body(tgt_ref, x_ref, out_ref, m_ref, se_ref, ssum_ref, pt_ref,
          *, C, B, R, V):
    i = pl.program_id(0)
    j = pl.program_id(1)
    nv = pl.num_programs(1)

    x = x_ref[...]
    ones = jnp.ones((V, 1), jnp.float32)
    iota = jax.lax.broadcasted_iota(jnp.int32, (R, V), 1)

    @pl.when(j == 0)
    def _():
        m_ref[...] = jnp.full((R, 1), -jnp.inf, jnp.float32)
        se_ref[...] = jnp.zeros((R, 1), jnp.float32)
        ssum_ref[...] = jnp.zeros((R, 1), jnp.float32)
        pt_ref[...] = jnp.zeros((R, 1), jnp.float32)

    def update(xneg, xz):
        tile_max = jnp.max(xneg, axis=1, keepdims=True)
        m_old = m_ref[...]
        m_new = jnp.maximum(m_old, tile_max)
        m_ref[...] = m_new
        e = jnp.exp(xneg - m_new)
        se_ref[...] = se_ref[...] * jnp.exp(m_old - m_new) + _rowsum(e, ones)
        ssum_ref[...] = ssum_ref[...] + _rowsum(xz, ones)

    @pl.when(j < nv - 1)
    def _():
        update(x, x)

    @pl.when(j == nv - 1)
    def _():
        valid = iota < (C - j * V)
        update(jnp.where(valid, x, -jnp.inf), jnp.where(valid, x, 0.0))

    t_rel = tgt_ref[...] - j * V
    pt_ref[...] = pt_ref[...] + _rowsum(jnp.where(iota == t_rel, x, 0.0), ones)

    @pl.when(j == nv - 1)
    def _():
        lse = m_ref[...] + jnp.log(se_ref[...])
        s = _SMOOTHING / (C - 1)
        row_loss = -(s * (ssum_ref[...] - C * lse)
                     + (_CONFIDENCE - s) * (pt_ref[...] - lse))
        total = (jnp.sum(row_loss) * (1.0 / B)).reshape(1, 1)

        @pl.when(i == 0)
        def _():
            out_ref[...] = total

        @pl.when(i > 0)
        def _():
            out_ref[...] = out_ref[...] + total


def kernel(pred, target):
    B, C = pred.shape
    R = 256
    V = 2048
    nb = B // R
    nv = pl.cdiv(C, V)

    out = pl.pallas_call(
        functools.partial(_body, C=C, B=B, R=R, V=V),
        grid=(nb, nv),
        in_specs=[
            pl.BlockSpec((R, 1), lambda i, j: (i, 0)),
            pl.BlockSpec((R, V), lambda i, j: (i, j)),
        ],
        out_specs=pl.BlockSpec((1, 1), lambda i, j: (0, 0)),
        out_shape=jax.ShapeDtypeStruct((1, 1), jnp.float32),
        scratch_shapes=[
            pltpu.VMEM((R, 1), jnp.float32),
            pltpu.VMEM((R, 1), jnp.float32),
            pltpu.VMEM((R, 1), jnp.float32),
            pltpu.VMEM((R, 1), jnp.float32),
        ],
    )(target.reshape(B, 1).astype(jnp.int32), pred)
    return out[0, 0]


# speedup vs baseline: 1.0869x; 1.0869x over previous
import functools

import jax
import jax.numpy as jnp
from jax.experimental import pallas as pl
from jax.experimental.pallas import tpu as pltpu

_SMOOTHING = 0.1
_CONFIDENCE = 1.0 - _SMOOTHING


def _rowsum(x, ones):
    return jax.lax.dot_general(x, ones, (((1,), (0,)), ((), ())),
                               preferred_element_type=jnp.float32)


def _body(tgt_ref, x_ref, out_ref, m_ref, se_ref, ssum_ref, pt_ref,
          *, C, B, R, V):
    i = pl.program_id(0)
    j = pl.program_id(1)
    nv = pl.num_programs(1)

    x = x_ref[...]
    ones = jnp.ones((V, 1), jnp.float32)
    iota = jax.lax.broadcasted_iota(jnp.int32, (R, V), 1)

    @pl.when(j == 0)
    def _():
        m_ref[...] = jnp.full((R, 1), -jnp.inf, jnp.float32)
        se_ref[...] = jnp.zeros((R, 1), jnp.float32)
        ssum_ref[...] = jnp.zeros((R, 1), jnp.float32)
        pt_ref[...] = jnp.zeros((R, 1), jnp.float32)

    def update(xneg, xz):
        tile_max = jnp.max(xneg, axis=1, keepdims=True)
        m_old = m_ref[...]
        m_new = jnp.maximum(m_old, tile_max)
        m_ref[...] = m_new
        e = jnp.exp(xneg - m_new)
        se_ref[...] = (se_ref[...] * jnp.exp(m_old - m_new)
                       + jnp.sum(e, axis=1, keepdims=True))
        ssum_ref[...] = ssum_ref[...] + jnp.sum(xz, axis=1, keepdims=True)

    @pl.when(j < nv - 1)
    def _():
        update(x, x)

    @pl.when(j == nv - 1)
    def _():
        valid = iota < (C - j * V)
        update(jnp.where(valid, x, -jnp.inf), jnp.where(valid, x, 0.0))

    t_rel = tgt_ref[...] - j * V
    pt_ref[...] = pt_ref[...] + jnp.sum(
        jnp.where(iota == t_rel, x, 0.0), axis=1, keepdims=True)

    @pl.when(j == nv - 1)
    def _():
        lse = m_ref[...] + jnp.log(se_ref[...])
        s = _SMOOTHING / (C - 1)
        row_loss = -(s * (ssum_ref[...] - C * lse)
                     + (_CONFIDENCE - s) * (pt_ref[...] - lse))
        total = (jnp.sum(row_loss) * (1.0 / B)).reshape(1, 1)

        @pl.when(i == 0)
        def _():
            out_ref[...] = total

        @pl.when(i > 0)
        def _():
            out_ref[...] = out_ref[...] + total


def kernel(pred, target):
    B, C = pred.shape
    R = 256
    V = 2048
    nb = B // R
    nv = pl.cdiv(C, V)

    out = pl.pallas_call(
        functools.partial(_body, C=C, B=B, R=R, V=V),
        grid=(nb, nv),
        in_specs=[
            pl.BlockSpec((R, 1), lambda i, j: (i, 0)),
            pl.BlockSpec((R, V), lambda i, j: (i, j)),
        ],
        out_specs=pl.BlockSpec((1, 1), lambda i, j: (0, 0)),
        out_shape=jax.ShapeDtypeStruct((1, 1), jnp.float32),
        scratch_shapes=[
            pltpu.VMEM((R, 1), jnp.float32),
            pltpu.VMEM((R, 1), jnp.float32),
            pltpu.VMEM((R, 1), jnp.float32),
            pltpu.VMEM((R, 1), jnp.float32),
        ],
    )(target.reshape(B, 1).astype(jnp.int32), pred)
    return out[0, 0]
